# baseline (speedup 1.0000x reference)
import functools

import jax
import jax.numpy as jnp
from jax import lax
from jax.experimental import pallas as pl
from jax.experimental.pallas import tpu as pltpu
from jax.experimental.pallas import tpu_sc as plsc

_NUM_CORES = 2
_NUM_SUBCORES = 16
_NW = _NUM_CORES * _NUM_SUBCORES
_R = 32
_GPR = 13
_NBUF = 4
_PITCH = 208

_TAKE_DNUMS = lax.GatherDimensionNumbers(
    offset_dims=(), collapsed_slice_dims=(0,), start_index_map=(0,))


def _lane_take(vec, lanes):
  return lax.gather(vec, lanes[:, None], _TAKE_DNUMS, (1,),
                    mode=lax.GatherScatterMode.PROMISE_IN_BOUNDS)


@functools.lru_cache(maxsize=None)
def _make_gather(b: int, m: int, d: int):
  assert d == 200, "granule geometry below is specific to 200-word rows"
  assert b % (_NW * _R) == 0
  b_per_w = b // _NW
  n_chunks = b_per_w // _R
  n_gran = m * d // 16
  mesh = plsc.VectorSubcoreMesh(core_axis_name="c", subcore_axis_name="s")

  @functools.partial(
      pl.kernel,
      mesh=mesh,
      out_type=jax.ShapeDtypeStruct((b, d), jnp.float32),
      compiler_params=pltpu.CompilerParams(use_tc_tiling_on_sc=False,
                                           needs_layout_passes=False),
      scratch_types=[
          pltpu.VMEM((b_per_w,), jnp.int32),
          pltpu.VMEM((_NBUF, 4, (_R // 4) * _GPR), jnp.int32),
          pltpu.VMEM((_NBUF, _R * _GPR, 16), jnp.float32),
          pltpu.VMEM((2, _R, _PITCH), jnp.float32),
      ] + [pltpu.SemaphoreType.DMA] * (_NBUF + 2),
  )
  def gather_kernel(table_hbm, idx_hbm, out_hbm, idx_v, gidx_v, win_v,
                    stg_v, *sems):
    wid = lax.axis_index("s") * _NUM_CORES + lax.axis_index("c")
    base = wid * b_per_w
    gsems = sems[:_NBUF]
    osems = sems[_NBUF:_NBUF + 2]
    tv = table_hbm
    iota = lax.iota(jnp.int32, 16)
    shf = (iota + 8) & 15

    pltpu.sync_copy(idx_hbm.at[pl.ds(base, b_per_w)], idx_v)

    def build_gidx(c, buf):
      for v in range(_R // 16):
        i16 = idx_v[pl.ds(c * _R + v * 16, 16)]
        g0 = (i16 * 25) >> 1
        rows = iota + (v * 16)
        sub = rows >> 3
        pos = (rows & 7) * _GPR
        for k in range(_GPR):
          plsc.store_scatter(gidx_v.at[buf], [sub, pos + k], g0 + k)

    def start_gather(buf):
      for t in range(4):
        pltpu.async_copy(tv.at[gidx_v.at[buf, t]],
                         win_v.at[buf, pl.ds(t * 8 * _GPR, 8 * _GPR)],
                         gsems[buf])

    def wait_gather(buf):
      for t in range(4):
        pltpu.make_async_copy(tv.at[gidx_v.at[buf, t]],
                              win_v.at[buf, pl.ds(t * 8 * _GPR, 8 * _GPR)],
                              gsems[buf]).wait()

    def repack(c, buf, sb):
      for v in range(_R // 16):
        par16 = (idx_v[pl.ds(c * _R + v * 16, 16)] & 1) << 3

        @pl.loop(0, 16)
        def _(r16):
          r = v * 16 + r16
          g = r * _GPR
          o = _lane_take(par16, jnp.full((16,), r16, jnp.int32))
          sh = (iota + o) & 15
          mask = (iota + o) < 16
          prev = win_v[buf, g, :]
          for k in range(_GPR - 1):
            nxt = win_v[buf, g + k + 1, :]
            ta = _lane_take(prev, sh)
            tb = _lane_take(nxt, sh)
            stg_v[sb, r, pl.ds(16 * k, 16)] = jnp.where(mask, ta, tb)
            prev = nxt
          stg_v[sb, r, pl.ds(16 * (_GPR - 1), 16)] = _lane_take(prev, sh)

    def start_out(c, sb):
      pltpu.async_copy(stg_v.at[sb, :, pl.ds(0, d)],
                       out_hbm.at[pl.ds(base + c * _R, _R)], osems[sb])

    def wait_out(sb):
      pltpu.make_async_copy(stg_v.at[sb, :, pl.ds(0, d)],
                            out_hbm.at[pl.ds(base, _R)], osems[sb]).wait()

    for c in range(_NBUF):
      build_gidx(c, c)
      start_gather(c)

    @pl.loop(0, n_chunks, step=_NBUF)
    def _(g):
      for j in range(_NBUF):
        c = g + j
        buf = j
        sb = j % 2
        wait_gather(buf)

        @pl.when(c >= 2)
        def _():
          wait_out(sb)

        repack(c, buf, sb)
        start_out(c, sb)

        @pl.when(c + _NBUF < n_chunks)
        def _():
          build_gidx(c + _NBUF, buf)
          start_gather(buf)

    wait_out(0)
    wait_out(1)

  return gather_kernel


def kernel(x, table):
  b = x.shape[0] * x.shape[1]
  m, d = table.shape
  idx = x.reshape((b,)).astype(jnp.int32)
  tv = table.reshape((m * d // 16, 16))
  out = _make_gather(b, m, d)(tv, idx)
  return out.reshape(x.shape + (d,))


# speedup vs baseline: 1.1109x; 1.1109x over previous
import functools

import jax
import jax.numpy as jnp
from jax import lax
from jax.experimental import pallas as pl
from jax.experimental.pallas import tpu as pltpu
from jax.experimental.pallas import tpu_sc as plsc

_NUM_CORES = 2
_NUM_SUBCORES = 16
_NW = _NUM_CORES * _NUM_SUBCORES
_OPC = 4
_NBUF = 2


@functools.lru_cache(maxsize=None)
def _make_gather(bi, bj, m, d):
  assert bi % (_NW * _OPC) == 0
  o_per_w = bi // _NW
  n_chunks = o_per_w // _OPC
  mesh = plsc.VectorSubcoreMesh(core_axis_name="c", subcore_axis_name="s")

  @functools.partial(
      pl.kernel,
      mesh=mesh,
      out_type=jax.ShapeDtypeStruct((bi, bj, d), jnp.float32),
      compiler_params=pltpu.CompilerParams(use_tc_tiling_on_sc=False),
      scratch_types=[
          pltpu.VMEM((_NBUF, _OPC, bj), jnp.int32),
          pltpu.VMEM((_NBUF, _OPC, bj, d), jnp.float32),
      ] + [pltpu.SemaphoreType.DMA] * (3 * _NBUF),
  )
  def gather_kernel(table_hbm, x_hbm, out_hbm, idx_v, rows_v, *sems):
    wid = lax.axis_index("s") * _NUM_CORES + lax.axis_index("c")
    obase = wid * o_per_w
    isems = sems[0:_NBUF]
    gsems = sems[_NBUF:2 * _NBUF]
    osems = sems[2 * _NBUF:]

    def start_idx(c, b):
      pltpu.async_copy(x_hbm.at[pl.ds(obase + c * _OPC, _OPC)],
                       idx_v.at[b], isems[b])

    def wait_idx(b):
      pltpu.make_async_copy(x_hbm.at[pl.ds(0, _OPC)], idx_v.at[b],
                            isems[b]).wait()

    def start_gather(b):
      for k in range(_OPC):
        pltpu.async_copy(table_hbm.at[idx_v.at[b, k]], rows_v.at[b, k],
                         gsems[b])

    def wait_gather(b):
      for k in range(_OPC):
        pltpu.make_async_copy(table_hbm.at[idx_v.at[b, k]], rows_v.at[b, k],
                              gsems[b]).wait()

    def start_out(c, b):
      pltpu.async_copy(rows_v.at[b], out_hbm.at[pl.ds(obase + c * _OPC, _OPC)],
                       osems[b])

    def wait_out(b):
      pltpu.make_async_copy(rows_v.at[b], out_hbm.at[pl.ds(0, _OPC)],
                            osems[b]).wait()

    start_idx(0, 0)
    start_idx(1, 1)
    wait_idx(0)
    start_gather(0)
    wait_idx(1)
    start_gather(1)

    @pl.loop(0, n_chunks, step=_NBUF)
    def _(g):
      for b in range(_NBUF):
        c = g + b
        wait_gather(b)
        start_out(c, b)

        @pl.when(c + _NBUF < n_chunks)
        def _():
          start_idx(c + _NBUF, b)
          wait_idx(b)
          wait_out(b)
          start_gather(b)

    wait_out(0)
    wait_out(1)

  return gather_kernel


def kernel(x, table):
  bi, bj = x.shape
  m, d = table.shape
  return _make_gather(bi, bj, m, d)(table, x.astype(jnp.int32))


# speedup vs baseline: 1.1113x; 1.0004x over previous
import functools

import jax
import jax.numpy as jnp
from jax import lax
from jax.experimental import pallas as pl
from jax.experimental.pallas import tpu as pltpu
from jax.experimental.pallas import tpu_sc as plsc

_NUM_CORES = 2
_NUM_SUBCORES = 16
_NW = _NUM_CORES * _NUM_SUBCORES
_OPC = 4
_NBUF = 2


@functools.lru_cache(maxsize=None)
def _make_gather(bi, bj, m, d):
  assert bi % (_NW * _OPC) == 0
  o_per_w = bi // _NW
  n_chunks = o_per_w // _OPC
  mesh = plsc.VectorSubcoreMesh(core_axis_name="c", subcore_axis_name="s")

  @functools.partial(
      pl.kernel,
      mesh=mesh,
      out_type=jax.ShapeDtypeStruct((bi, bj, d), jnp.float32),
      compiler_params=pltpu.CompilerParams(use_tc_tiling_on_sc=False,
                                           skip_device_barrier=True),
      scratch_types=[
          pltpu.VMEM((_NBUF, _OPC, bj), jnp.int32),
          pltpu.VMEM((_NBUF, _OPC, bj, d), jnp.float32),
      ] + [pltpu.SemaphoreType.DMA] * (3 * _NBUF),
  )
  def gather_kernel(table_hbm, x_hbm, out_hbm, idx_v, rows_v, *sems):
    wid = lax.axis_index("s") * _NUM_CORES + lax.axis_index("c")
    obase = wid * o_per_w
    isems = sems[0:_NBUF]
    gsems = sems[_NBUF:2 * _NBUF]
    osems = sems[2 * _NBUF:]

    def start_idx(c, b):
      pltpu.async_copy(x_hbm.at[pl.ds(obase + c * _OPC, _OPC)],
                       idx_v.at[b], isems[b])

    def wait_idx(b):
      pltpu.make_async_copy(x_hbm.at[pl.ds(0, _OPC)], idx_v.at[b],
                            isems[b]).wait()

    def start_gather(b):
      for k in range(_OPC):
        pltpu.async_copy(table_hbm.at[idx_v.at[b, k]], rows_v.at[b, k],
                         gsems[b])

    def wait_gather(b):
      for k in range(_OPC):
        pltpu.make_async_copy(table_hbm.at[idx_v.at[b, k]], rows_v.at[b, k],
                              gsems[b]).wait()

    def start_out(c, b):
      pltpu.async_copy(rows_v.at[b], out_hbm.at[pl.ds(obase + c * _OPC, _OPC)],
                       osems[b])

    def wait_out(b):
      pltpu.make_async_copy(rows_v.at[b], out_hbm.at[pl.ds(0, _OPC)],
                            osems[b]).wait()

    start_idx(0, 0)
    start_idx(1, 1)
    wait_idx(0)
    start_gather(0)
    wait_idx(1)
    start_gather(1)

    @pl.loop(0, n_chunks, step=_NBUF)
    def _(g):
      for b in range(_NBUF):
        c = g + b
        wait_gather(b)
        start_out(c, b)

        @pl.when(c + _NBUF < n_chunks)
        def _():
          start_idx(c + _NBUF, b)
          wait_idx(b)
          wait_out(b)
          start_gather(b)

    wait_out(0)
    wait_out(1)

  return gather_kernel


def kernel(x, table):
  bi, bj = x.shape
  m, d = table.shape
  return _make_gather(bi, bj, m, d)(table, x.astype(jnp.int32))
